# speedup vs baseline: 3.2162x; 3.2162x over previous
import functools

import jax
import jax.numpy as jnp
from jax import lax
from jax.experimental import pallas as pl
from jax.experimental.pallas import tpu as pltpu

N = 10000
E = 320000
B = 256
IN = 128
D = 350
H1 = 64
H2 = 32
NC = 10

BN = 1000
GRID_N = N // BN


def _tc1_body(sx, Wv, bv, W1l, b1l, W1r, b1r, xl_o, xr_o):
    h0 = jnp.dot(sx[...], Wv[...], preferred_element_type=jnp.float32) + bv[...]
    xl_o[...] = jnp.dot(h0, W1l[...], preferred_element_type=jnp.float32) + b1l[...]
    xr_o[...] = jnp.dot(h0, W1r[...], preferred_element_type=jnp.float32) + b1r[...]


def _tc1(s_x, Wv, bv, W1l, b1l, W1r, b1r):
    full = lambda shape: pl.BlockSpec(shape, lambda i: tuple(0 for _ in shape))
    return pl.pallas_call(
        _tc1_body,
        grid=(GRID_N,),
        in_specs=[
            pl.BlockSpec((BN, IN), lambda i: (i, 0)),
            full((IN, D)), full((1, D)),
            full((D, H1)), full((1, H1)),
            full((D, H1)), full((1, H1)),
        ],
        out_specs=[
            pl.BlockSpec((BN, H1), lambda i: (i, 0)),
            pl.BlockSpec((BN, H1), lambda i: (i, 0)),
        ],
        out_shape=[
            jax.ShapeDtypeStruct((N, H1), jnp.float32),
            jax.ShapeDtypeStruct((N, H1), jnp.float32),
        ],
    )(s_x, Wv, bv.reshape(1, D), W1l, b1l.reshape(1, H1), W1r, b1r.reshape(1, H1))


def _gat_epilogue(xl, xr, acc, denp, att, bias):
    t = xl + xr
    lr = jnp.maximum(t, 0.2 * t)
    e = jnp.sum(lr * att, axis=1, keepdims=True)
    es = jnp.exp(e)
    den = jnp.sum(denp, axis=0) + es
    accs = acc[0] + acc[1] + es * xl
    return jnp.maximum(accs / den + bias, 0.0)


def _tcmid_body(xl, xr, acc, denp, att, bias, Wl, bl, Wr, br, xl_o, xr_o):
    h = _gat_epilogue(xl[...], xr[...], acc[...], denp[...], att[...], bias[...])
    xl_o[...] = jnp.dot(h, Wl[...], preferred_element_type=jnp.float32) + bl[...]
    xr_o[...] = jnp.dot(h, Wr[...], preferred_element_type=jnp.float32) + br[...]


def _tcmid(xl, xr, acc, denp, att, bias, Wl, bl, Wr, br, Hp, Hn):
    full = lambda shape: pl.BlockSpec(shape, lambda i: tuple(0 for _ in shape))
    return pl.pallas_call(
        _tcmid_body,
        grid=(GRID_N,),
        in_specs=[
            pl.BlockSpec((BN, Hp), lambda i: (i, 0)),
            pl.BlockSpec((BN, Hp), lambda i: (i, 0)),
            pl.BlockSpec((2, BN, Hp), lambda i: (0, i, 0)),
            pl.BlockSpec((32, BN, 1), lambda i: (0, i, 0)),
            full((1, Hp)), full((1, Hp)),
            full((Hp, Hn)), full((1, Hn)),
            full((Hp, Hn)), full((1, Hn)),
        ],
        out_specs=[
            pl.BlockSpec((BN, Hn), lambda i: (i, 0)),
            pl.BlockSpec((BN, Hn), lambda i: (i, 0)),
        ],
        out_shape=[
            jax.ShapeDtypeStruct((N, Hn), jnp.float32),
            jax.ShapeDtypeStruct((N, Hn), jnp.float32),
        ],
    )(xl, xr, acc, denp.reshape(32, N, 1), att.reshape(1, Hp), bias.reshape(1, Hp),
      Wl, bl.reshape(1, Hn), Wr, br.reshape(1, Hn))


def _tc4_body(xl, xr, acc, denp, att, bias, batch, root, sx,
              sums_o, cnt_o, hroot_o, sxroot_o):
    i = pl.program_id(0)
    h = _gat_epilogue(xl[...], xr[...], acc[...], denp[...], att[...], bias[...])
    rows = lax.broadcasted_iota(jnp.int32, (1, BN), 1) + i * BN
    seg = lax.broadcasted_iota(jnp.int32, (B, 1), 0)
    bmask = (seg == batch[0]).astype(jnp.float32)
    rmask = (jnp.transpose(root[...]) == rows).astype(jnp.float32)
    sums_c = jnp.dot(bmask, h, preferred_element_type=jnp.float32)
    cnt_c = jnp.sum(bmask, axis=1, keepdims=True)
    hroot_c = jnp.dot(rmask, h, preferred_element_type=jnp.float32)
    sxroot_c = jnp.dot(rmask, sx[...], preferred_element_type=jnp.float32)

    @pl.when(i == 0)
    def _():
        sums_o[...] = sums_c
        cnt_o[...] = cnt_c
        hroot_o[...] = hroot_c
        sxroot_o[...] = sxroot_c

    @pl.when(i > 0)
    def _():
        sums_o[...] += sums_c
        cnt_o[...] += cnt_c
        hroot_o[...] += hroot_c
        sxroot_o[...] += sxroot_c


def _tc4(xl, xr, acc, denp, att, bias, batch, root, s_x):
    full = lambda shape: pl.BlockSpec(shape, lambda i: tuple(0 for _ in shape))
    H = H2
    return pl.pallas_call(
        _tc4_body,
        grid=(GRID_N,),
        in_specs=[
            pl.BlockSpec((BN, H), lambda i: (i, 0)),
            pl.BlockSpec((BN, H), lambda i: (i, 0)),
            pl.BlockSpec((2, BN, H), lambda i: (0, i, 0)),
            pl.BlockSpec((32, BN, 1), lambda i: (0, i, 0)),
            full((1, H)), full((1, H)),
            pl.BlockSpec((1, 1, BN), lambda i: (i, 0, 0)),
            full((1, B)),
            pl.BlockSpec((BN, IN), lambda i: (i, 0)),
        ],
        out_specs=[full((B, H)), full((B, 1)), full((B, H)), full((B, IN))],
        out_shape=[
            jax.ShapeDtypeStruct((B, H), jnp.float32),
            jax.ShapeDtypeStruct((B, 1), jnp.float32),
            jax.ShapeDtypeStruct((B, H), jnp.float32),
            jax.ShapeDtypeStruct((B, IN), jnp.float32),
        ],
    )(xl, xr, acc, denp.reshape(32, N, 1), att.reshape(1, H), bias.reshape(1, H),
      batch.reshape(GRID_N, 1, BN), root.reshape(1, B), s_x)


def _tc5_body(sums, cnt, hroot, sxroot, cw, cb, c2W, c2b, c3W, c3b,
              linW, linb, aW1, ab1, aW2, mW1, mb1, mW2, mb2, out_o):
    gmp = sums[...] / jnp.maximum(cnt[...], 1.0)
    info = sxroot[...]
    y = (cw[0, 0:1] * info[:, 0:IN - 2] + cw[0, 1:2] * info[:, 1:IN - 1]
         + cw[0, 2:3] * info[:, 2:IN] + cb[...])
    z = jnp.maximum(jnp.dot(y, c2W[...], preferred_element_type=jnp.float32) + c2b[...], 0.0)
    z = jnp.maximum(jnp.dot(z, c3W[...], preferred_element_type=jnp.float32) + c3b[...], 0.0)
    s_info = z
    sx_cat = jnp.concatenate([hroot[...], gmp], axis=-1)
    sx2 = jnp.maximum(jnp.dot(sx_cat, linW[...], preferred_element_type=jnp.float32) + linb[...], 0.0)
    w1 = jnp.dot(jnp.tanh(jnp.dot(sx2, aW1[...], preferred_element_type=jnp.float32) + ab1[...]),
                 aW2[...], preferred_element_type=jnp.float32)
    w2 = jnp.dot(jnp.tanh(jnp.dot(s_info, aW1[...], preferred_element_type=jnp.float32) + ab1[...]),
                 aW2[...], preferred_element_type=jnp.float32)
    m = jnp.maximum(w1, w2)
    e1 = jnp.exp(w1 - m)
    e2 = jnp.exp(w2 - m)
    emb2 = (e1 * sx2 + e2 * s_info) / (e1 + e2)
    logits = (jnp.dot(jnp.tanh(jnp.dot(emb2, mW1[...], preferred_element_type=jnp.float32) + mb1[...]),
                      mW2[...], preferred_element_type=jnp.float32) + mb2[...])
    lm = jnp.max(logits, axis=1, keepdims=True)
    el = jnp.exp(logits - lm)
    out_o[...] = el / jnp.sum(el, axis=1, keepdims=True)


def _tc5(sums, cnt, hroot, sxroot, cnn1_w, cnn1_b, cnn2_W, cnn2_b, cnn3_W, cnn3_b,
         lin_W, lin_b, attW1, attb1, attW2, mlpW1, mlpb1, mlpW2, mlpb2):
    args = (sums, cnt, hroot, sxroot,
            cnn1_w.reshape(1, 3), cnn1_b.reshape(1, 1),
            jnp.transpose(cnn2_W), cnn2_b.reshape(1, H1),
            jnp.transpose(cnn3_W), cnn3_b.reshape(1, H2),
            lin_W, lin_b.reshape(1, H2),
            attW1, attb1.reshape(1, 16), attW2,
            mlpW1, mlpb1.reshape(1, 16), mlpW2, mlpb2.reshape(1, NC))
    return pl.pallas_call(
        _tc5_body,
        out_shape=jax.ShapeDtypeStruct((B, NC), jnp.float32),
    )(*args)


def _edges_jnp(xl, xr, att, src, dst, H):
    t = xl[src] + xr[dst]
    e = jnp.maximum(t, 0.2 * t) @ att
    ex = jnp.exp(e)
    den = jax.ops.segment_sum(ex, dst, num_segments=N)
    acc = jax.ops.segment_sum(ex[:, None] * xl[src], dst, num_segments=N)
    acc2 = jnp.stack([acc, jnp.zeros_like(acc)])
    denp = jnp.concatenate([den[None], jnp.zeros((31, N), jnp.float32)])
    return acc2, denp


def kernel(s_x, s_edge_index, s_batch, s_root_n_id, Wq, bq, Wk, bk, Wv, bv,
           g1_Wl, g1_bl, g1_Wr, g1_br, g1_att, g1_bias,
           g2_Wl, g2_bl, g2_Wr, g2_br, g2_att, g2_bias,
           g3_Wl, g3_bl, g3_Wr, g3_br, g3_att, g3_bias,
           cnn1_w, cnn1_b, cnn2_W, cnn2_b, cnn3_W, cnn3_b,
           lin_W, lin_b, attW1, attb1, attW2,
           mlpW1, mlpb1, mlpW2, mlpb2):
    src = s_edge_index[0]
    dst = s_edge_index[1]

    xl1, xr1 = _tc1(s_x, Wv, bv, g1_Wl, g1_bl, g1_Wr, g1_br)
    acc1, denp1 = _edges_jnp(xl1, xr1, g1_att, src, dst, H1)
    xl2, xr2 = _tcmid(xl1, xr1, acc1, denp1, g1_att, g1_bias,
                      g2_Wl, g2_bl, g2_Wr, g2_br, H1, H2)
    acc2, denp2 = _edges_jnp(xl2, xr2, g2_att, src, dst, H2)
    xl3, xr3 = _tcmid(xl2, xr2, acc2, denp2, g2_att, g2_bias,
                      g3_Wl, g3_bl, g3_Wr, g3_br, H2, H2)
    acc3, denp3 = _edges_jnp(xl3, xr3, g3_att, src, dst, H2)
    sums, cnt, hroot, sxroot = _tc4(xl3, xr3, acc3, denp3, g3_att, g3_bias,
                                    s_batch, s_root_n_id, s_x)
    return _tc5(sums, cnt, hroot, sxroot, cnn1_w, cnn1_b, cnn2_W, cnn2_b,
                cnn3_W, cnn3_b, lin_W, lin_b, attW1, attb1, attW2,
                mlpW1, mlpb1, mlpW2, mlpb2)


# speedup vs baseline: 7.3015x; 2.2702x over previous
import functools

import jax
import jax.numpy as jnp
from jax import lax
from jax.experimental import pallas as pl
from jax.experimental.pallas import tpu as pltpu
from jax.experimental.pallas import tpu_sc as plsc

N = 10000
E = 320000
B = 256
IN = 128
D = 350
H1 = 64
H2 = 32
NC = 10

BN = 1000
GRID_N = N // BN


def _tc1_body(sx, Wv, bv, W1l, b1l, W1r, b1r, xl_o, xr_o):
    h0 = jnp.dot(sx[...], Wv[...], preferred_element_type=jnp.float32) + bv[...]
    xl_o[...] = jnp.dot(h0, W1l[...], preferred_element_type=jnp.float32) + b1l[...]
    xr_o[...] = jnp.dot(h0, W1r[...], preferred_element_type=jnp.float32) + b1r[...]


def _tc1(s_x, Wv, bv, W1l, b1l, W1r, b1r):
    full = lambda shape: pl.BlockSpec(shape, lambda i: tuple(0 for _ in shape))
    return pl.pallas_call(
        _tc1_body,
        grid=(GRID_N,),
        in_specs=[
            pl.BlockSpec((BN, IN), lambda i: (i, 0)),
            full((IN, D)), full((1, D)),
            full((D, H1)), full((1, H1)),
            full((D, H1)), full((1, H1)),
        ],
        out_specs=[
            pl.BlockSpec((BN, H1), lambda i: (i, 0)),
            pl.BlockSpec((BN, H1), lambda i: (i, 0)),
        ],
        out_shape=[
            jax.ShapeDtypeStruct((N, H1), jnp.float32),
            jax.ShapeDtypeStruct((N, H1), jnp.float32),
        ],
    )(s_x, Wv, bv.reshape(1, D), W1l, b1l.reshape(1, H1), W1r, b1r.reshape(1, H1))


def _gat_epilogue(xl, xr, acc, denp, att, bias):
    t = xl + xr
    lr = jnp.maximum(t, 0.2 * t)
    e = jnp.sum(lr * att, axis=1, keepdims=True)
    es = jnp.exp(e)
    den = jnp.sum(denp, axis=0) + es
    accs = acc[0] + acc[1] + es * xl
    return jnp.maximum(accs / den + bias, 0.0)


def _tcmid_body(xl, xr, acc, denp, att, bias, Wl, bl, Wr, br, xl_o, xr_o):
    h = _gat_epilogue(xl[...], xr[...], acc[...], denp[...], att[...], bias[...])
    xl_o[...] = jnp.dot(h, Wl[...], preferred_element_type=jnp.float32) + bl[...]
    xr_o[...] = jnp.dot(h, Wr[...], preferred_element_type=jnp.float32) + br[...]


def _tcmid(xl, xr, acc, denp, att, bias, Wl, bl, Wr, br, Hp, Hn):
    full = lambda shape: pl.BlockSpec(shape, lambda i: tuple(0 for _ in shape))
    return pl.pallas_call(
        _tcmid_body,
        grid=(GRID_N,),
        in_specs=[
            pl.BlockSpec((BN, Hp), lambda i: (i, 0)),
            pl.BlockSpec((BN, Hp), lambda i: (i, 0)),
            pl.BlockSpec((2, BN, Hp), lambda i: (0, i, 0)),
            pl.BlockSpec((32, BN, 1), lambda i: (0, i, 0)),
            full((1, Hp)), full((1, Hp)),
            full((Hp, Hn)), full((1, Hn)),
            full((Hp, Hn)), full((1, Hn)),
        ],
        out_specs=[
            pl.BlockSpec((BN, Hn), lambda i: (i, 0)),
            pl.BlockSpec((BN, Hn), lambda i: (i, 0)),
        ],
        out_shape=[
            jax.ShapeDtypeStruct((N, Hn), jnp.float32),
            jax.ShapeDtypeStruct((N, Hn), jnp.float32),
        ],
    )(xl, xr, acc, denp.reshape(32, N, 1), att.reshape(1, Hp), bias.reshape(1, Hp),
      Wl, bl.reshape(1, Hn), Wr, br.reshape(1, Hn))


def _tc4_body(xl, xr, acc, denp, att, bias, batch, root, sx,
              sums_o, cnt_o, hroot_o, sxroot_o):
    i = pl.program_id(0)
    h = _gat_epilogue(xl[...], xr[...], acc[...], denp[...], att[...], bias[...])
    rows = lax.broadcasted_iota(jnp.int32, (1, BN), 1) + i * BN
    seg = lax.broadcasted_iota(jnp.int32, (B, 1), 0)
    bmask = (seg == batch[0]).astype(jnp.float32)
    rmask = (jnp.transpose(root[...]) == rows).astype(jnp.float32)
    sums_c = jnp.dot(bmask, h, preferred_element_type=jnp.float32)
    cnt_c = jnp.sum(bmask, axis=1, keepdims=True)
    hroot_c = jnp.dot(rmask, h, preferred_element_type=jnp.float32)
    sxroot_c = jnp.dot(rmask, sx[...], preferred_element_type=jnp.float32)

    @pl.when(i == 0)
    def _():
        sums_o[...] = sums_c
        cnt_o[...] = cnt_c
        hroot_o[...] = hroot_c
        sxroot_o[...] = sxroot_c

    @pl.when(i > 0)
    def _():
        sums_o[...] += sums_c
        cnt_o[...] += cnt_c
        hroot_o[...] += hroot_c
        sxroot_o[...] += sxroot_c


def _tc4(xl, xr, acc, denp, att, bias, batch, root, s_x):
    full = lambda shape: pl.BlockSpec(shape, lambda i: tuple(0 for _ in shape))
    H = H2
    return pl.pallas_call(
        _tc4_body,
        grid=(GRID_N,),
        in_specs=[
            pl.BlockSpec((BN, H), lambda i: (i, 0)),
            pl.BlockSpec((BN, H), lambda i: (i, 0)),
            pl.BlockSpec((2, BN, H), lambda i: (0, i, 0)),
            pl.BlockSpec((32, BN, 1), lambda i: (0, i, 0)),
            full((1, H)), full((1, H)),
            pl.BlockSpec((1, 1, BN), lambda i: (i, 0, 0)),
            full((1, B)),
            pl.BlockSpec((BN, IN), lambda i: (i, 0)),
        ],
        out_specs=[full((B, H)), full((B, 1)), full((B, H)), full((B, IN))],
        out_shape=[
            jax.ShapeDtypeStruct((B, H), jnp.float32),
            jax.ShapeDtypeStruct((B, 1), jnp.float32),
            jax.ShapeDtypeStruct((B, H), jnp.float32),
            jax.ShapeDtypeStruct((B, IN), jnp.float32),
        ],
    )(xl, xr, acc, denp.reshape(32, N, 1), att.reshape(1, H), bias.reshape(1, H),
      batch.reshape(GRID_N, 1, BN), root.reshape(1, B), s_x)


def _tc5_body(sums, cnt, hroot, sxroot, cw, cb, c2W, c2b, c3W, c3b,
              linW, linb, aW1, ab1, aW2, mW1, mb1, mW2, mb2, out_o):
    gmp = sums[...] / jnp.maximum(cnt[...], 1.0)
    info = sxroot[...]
    y = (cw[0, 0:1] * info[:, 0:IN - 2] + cw[0, 1:2] * info[:, 1:IN - 1]
         + cw[0, 2:3] * info[:, 2:IN] + cb[...])
    z = jnp.maximum(jnp.dot(y, c2W[...], preferred_element_type=jnp.float32) + c2b[...], 0.0)
    z = jnp.maximum(jnp.dot(z, c3W[...], preferred_element_type=jnp.float32) + c3b[...], 0.0)
    s_info = z
    sx_cat = jnp.concatenate([hroot[...], gmp], axis=-1)
    sx2 = jnp.maximum(jnp.dot(sx_cat, linW[...], preferred_element_type=jnp.float32) + linb[...], 0.0)
    w1 = jnp.dot(jnp.tanh(jnp.dot(sx2, aW1[...], preferred_element_type=jnp.float32) + ab1[...]),
                 aW2[...], preferred_element_type=jnp.float32)
    w2 = jnp.dot(jnp.tanh(jnp.dot(s_info, aW1[...], preferred_element_type=jnp.float32) + ab1[...]),
                 aW2[...], preferred_element_type=jnp.float32)
    m = jnp.maximum(w1, w2)
    e1 = jnp.exp(w1 - m)
    e2 = jnp.exp(w2 - m)
    emb2 = (e1 * sx2 + e2 * s_info) / (e1 + e2)
    logits = (jnp.dot(jnp.tanh(jnp.dot(emb2, mW1[...], preferred_element_type=jnp.float32) + mb1[...]),
                      mW2[...], preferred_element_type=jnp.float32) + mb2[...])
    lm = jnp.max(logits, axis=1, keepdims=True)
    el = jnp.exp(logits - lm)
    out_o[...] = el / jnp.sum(el, axis=1, keepdims=True)


def _tc5(sums, cnt, hroot, sxroot, cnn1_w, cnn1_b, cnn2_W, cnn2_b, cnn3_W, cnn3_b,
         lin_W, lin_b, attW1, attb1, attW2, mlpW1, mlpb1, mlpW2, mlpb2):
    args = (sums, cnt, hroot, sxroot,
            cnn1_w.reshape(1, 3), cnn1_b.reshape(1, 1),
            jnp.transpose(cnn2_W), cnn2_b.reshape(1, H1),
            jnp.transpose(cnn3_W), cnn3_b.reshape(1, H2),
            lin_W, lin_b.reshape(1, H2),
            attW1, attb1.reshape(1, 16), attW2,
            mlpW1, mlpb1.reshape(1, 16), mlpW2, mlpb2.reshape(1, NC))
    return pl.pallas_call(
        _tc5_body,
        out_shape=jax.ShapeDtypeStruct((B, NC), jnp.float32),
    )(*args)


EK = 80
TILES = 32
EPT = E // TILES
NBLK = EPT // EK
NP = 10240
NPT = NP // 16


def _sc_edge_body(H, xl_hbm, xr_hbm, src_hbm, dst_hbm, att_hbm, znh_hbm, zn_hbm,
                  acc_out, den_out,
                  src_v, dst_v, xlg, xrg, sbuf, exbuf, den_local, att_v,
                  acc_sh, sem1, sem2):
    c = lax.axis_index("c")
    s = lax.axis_index("s")
    base_e = (c * 16 + s) * EPT
    iota16 = jnp.arange(16, dtype=jnp.int32)

    pltpu.sync_copy(znh_hbm.at[pl.ds(s * NPT, NPT)], acc_sh.at[pl.ds(s * NPT, NPT)])
    pltpu.sync_copy(zn_hbm, den_local)
    pltpu.sync_copy(att_hbm, att_v)
    plsc.subcore_barrier()

    def blk_body(blk, carry):
        base = base_e + blk * EK
        pltpu.sync_copy(src_hbm.at[pl.ds(base, EK)], src_v)
        pltpu.sync_copy(dst_hbm.at[pl.ds(base, EK)], dst_v)
        cp1 = pltpu.async_copy(xl_hbm.at[src_v], xlg, sem1)
        cp2 = pltpu.async_copy(xr_hbm.at[dst_v], xrg, sem2)
        cp1.wait()
        cp2.wait()

        def h_body(h, accs):
            hvec = jnp.full((16,), h, dtype=jnp.int32)
            att_s = plsc.load_gather(att_v, [hvec])
            new = []
            for g in range(EK // 16):
                rows = iota16 + (g * 16)
                a = plsc.load_gather(xlg, [rows, hvec])
                b = plsc.load_gather(xrg, [rows, hvec])
                t = a + b
                t = jnp.maximum(t, 0.2 * t)
                new.append(accs[g] + att_s * t)
            return tuple(new)

        accs = lax.fori_loop(0, H, h_body,
                             tuple(jnp.zeros((16,), jnp.float32)
                                   for _ in range(EK // 16)))
        for g in range(EK // 16):
            ex_g = jnp.exp(accs[g])
            exbuf[pl.ds(g * 16, 16)] = ex_g
            dst_g = dst_v[pl.ds(g * 16, 16)]
            plsc.addupdate_scatter(den_local,
                                   [lax.shift_right_logical(dst_g, 4),
                                    lax.bitwise_and(dst_g, 15)], ex_g)

        def j_body(j, carry2):
            jvec = jnp.full((16,), j, dtype=jnp.int32)
            es = plsc.load_gather(exbuf, [jvec])
            for k2 in range(H // 16):
                cols = iota16 + (k2 * 16)
                row = plsc.load_gather(xlg, [jvec, cols])
                plsc.store_scatter(sbuf, [jvec, cols], row * es)
            return carry2

        lax.fori_loop(0, EK, j_body, 0)
        pltpu.sync_copy(sbuf, acc_sh.at[dst_v], add=True)
        return carry

    lax.fori_loop(0, NBLK, blk_body, 0)
    plsc.subcore_barrier()

    pltpu.sync_copy(acc_sh.at[pl.ds(s * NPT, NPT)],
                    acc_out.at[c].at[pl.ds(s * NPT, NPT)])
    pltpu.sync_copy(den_local, den_out.at[c].at[s])


def _sc_edges(xl, xr, att, src, dst, H):
    mesh = plsc.VectorSubcoreMesh(core_axis_name="c", subcore_axis_name="s")
    znh = jnp.zeros((NP, H), jnp.float32)
    zn = jnp.zeros((N // 16, 16), jnp.float32)
    kfn = functools.partial(
        pl.kernel,
        mesh=mesh,
        compiler_params=pltpu.CompilerParams(use_tc_tiling_on_sc=False, needs_layout_passes=False),
        out_type=[
            jax.ShapeDtypeStruct((2, NP, H), jnp.float32),
            jax.ShapeDtypeStruct((2, 16, N // 16, 16), jnp.float32),
        ],
        scratch_types=[
            pltpu.VMEM((EK,), jnp.int32),
            pltpu.VMEM((EK,), jnp.int32),
            pltpu.VMEM((EK, H), jnp.float32),
            pltpu.VMEM((EK, H), jnp.float32),
            pltpu.VMEM((EK, H), jnp.float32),
            pltpu.VMEM((EK,), jnp.float32),
            pltpu.VMEM((N // 16, 16), jnp.float32),
            pltpu.VMEM((H,), jnp.float32),
            pltpu.VMEM_SHARED((NP, H), jnp.float32),
            pltpu.SemaphoreType.DMA,
            pltpu.SemaphoreType.DMA,
        ],
    )(functools.partial(_sc_edge_body, H))
    acc2, denp = kfn(xl, xr, src, dst, att, znh, zn)
    return acc2, denp.reshape(TILES, N)


def kernel(s_x, s_edge_index, s_batch, s_root_n_id, Wq, bq, Wk, bk, Wv, bv,
           g1_Wl, g1_bl, g1_Wr, g1_br, g1_att, g1_bias,
           g2_Wl, g2_bl, g2_Wr, g2_br, g2_att, g2_bias,
           g3_Wl, g3_bl, g3_Wr, g3_br, g3_att, g3_bias,
           cnn1_w, cnn1_b, cnn2_W, cnn2_b, cnn3_W, cnn3_b,
           lin_W, lin_b, attW1, attb1, attW2,
           mlpW1, mlpb1, mlpW2, mlpb2):
    src = s_edge_index[0]
    dst = s_edge_index[1]

    xl1, xr1 = _tc1(s_x, Wv, bv, g1_Wl, g1_bl, g1_Wr, g1_br)
    acc1, denp1 = _sc_edges(xl1, xr1, g1_att, src, dst, H1)
    xl2, xr2 = _tcmid(xl1, xr1, acc1, denp1, g1_att, g1_bias,
                      g2_Wl, g2_bl, g2_Wr, g2_br, H1, H2)
    acc2, denp2 = _sc_edges(xl2, xr2, g2_att, src, dst, H2)
    xl3, xr3 = _tcmid(xl2, xr2, acc2, denp2, g2_att, g2_bias,
                      g3_Wl, g3_bl, g3_Wr, g3_br, H2, H2)
    acc3, denp3 = _sc_edges(xl3, xr3, g3_att, src, dst, H2)
    sums, cnt, hroot, sxroot = _tc4(xl3, xr3, acc3, denp3, g3_att, g3_bias,
                                    s_batch, s_root_n_id, s_x)
    return _tc5(sums, cnt, hroot, sxroot, cnn1_w, cnn1_b, cnn2_W, cnn2_b,
                cnn3_W, cnn3_b, lin_W, lin_b, attW1, attb1, attW2,
                mlpW1, mlpb1, mlpW2, mlpb2)
